# Initial kernel scaffold; baseline (speedup 1.0000x reference)
import jax
import jax.numpy as jnp
from jax.experimental import pallas as pl


def kernel(x, y, x_mask, actions, weight, bias):
    raise NotImplementedError("write your pallas kernel here")


# speedup vs baseline: 1.7714x; 1.7714x over previous
import jax
import jax.numpy as jnp
from jax.experimental import pallas as pl
from jax.experimental.pallas import tpu as pltpu


def _body(perm_ref, act_ref, x_ref, y_ref, mask_ref, w_ref, b_ref, out_ref):
    w = w_ref[0]
    yv = y_ref[0]
    wy = jax.lax.dot_general(
        yv, w, (((1,), (0,)), ((), ())),
        preferred_element_type=jnp.float32)
    wy = wy + b_ref[0]
    x = x_ref[0]
    s = jax.lax.dot_general(
        wy, x, (((1,), (1,)), ((), ())),
        preferred_element_type=jnp.float32)
    s = jnp.where(mask_ref[0] != 0, -jnp.inf, s)
    m = jnp.max(s, axis=-1, keepdims=True)
    sh = s - m
    lse = jnp.log(jnp.sum(jnp.exp(sh), axis=-1, keepdims=True))
    out_ref[0] = sh - lse


def kernel(x, y, x_mask, actions, weight, bias):
    B, L, X = x.shape
    A, Y, _ = weight.shape
    actions = actions.astype(jnp.int32)
    perm = jnp.argsort(actions).astype(jnp.int32)
    sorted_act = jnp.take(actions, perm)
    mask_i32 = x_mask.astype(jnp.int32).reshape(B, 1, L)
    y3 = y.reshape(B, 1, Y)
    bias3 = bias.reshape(A, 1, X)

    grid_spec = pltpu.PrefetchScalarGridSpec(
        num_scalar_prefetch=2,
        grid=(B,),
        in_specs=[
            pl.BlockSpec((1, L, X), lambda i, perm, act: (perm[i], 0, 0)),
            pl.BlockSpec((1, 1, Y), lambda i, perm, act: (perm[i], 0, 0)),
            pl.BlockSpec((1, 1, L), lambda i, perm, act: (perm[i], 0, 0)),
            pl.BlockSpec((1, Y, X), lambda i, perm, act: (act[i], 0, 0)),
            pl.BlockSpec((1, 1, X), lambda i, perm, act: (act[i], 0, 0)),
        ],
        out_specs=pl.BlockSpec((1, 1, L), lambda i, perm, act: (perm[i], 0, 0)),
    )
    out = pl.pallas_call(
        _body,
        grid_spec=grid_spec,
        out_shape=jax.ShapeDtypeStruct((B, 1, L), jnp.float32),
        compiler_params=pltpu.CompilerParams(
            dimension_semantics=("arbitrary",),
        ),
        name="bilinear_seq_attn",
    )(perm, sorted_act, x, y3, mask_i32, weight, bias3)
    return out.reshape(B, L)
